# baseline (speedup 1.0000x reference)
import functools

import jax
import jax.numpy as jnp
from jax import lax
from jax.experimental import pallas as pl
from jax.experimental.pallas import tpu as pltpu
from jax.experimental.pallas import tpu_sc as plsc

_NSTY = 256
_NCMP = 256
_B = 262144
_D = 128
_M = _NSTY * _NCMP

_NC, _NS = 2, 16
_CG = 16
_NG = _D // _CG
_GPC = _NG // _NC
_RPT = _B // _NS
_CHUNK = 1024
_NCHUNK = _RPT // _CHUNK
_STRIPE = _M // _NS
_ZCH = 512

_mesh = plsc.VectorSubcoreMesh(core_axis_name="c", subcore_axis_name="s")


@functools.partial(
    pl.kernel,
    out_type=[
        jax.ShapeDtypeStruct((_M, _D), jnp.float32),
        jax.ShapeDtypeStruct((_M, _CG), jnp.float32),
    ],
    mesh=_mesh,
    scratch_types=[
        pltpu.VMEM_SHARED((_M, _CG), jnp.float32),
        pltpu.VMEM((_CHUNK,), jnp.int32),
        pltpu.VMEM((_CHUNK,), jnp.int32),
        pltpu.VMEM((_CHUNK, _CG), jnp.float32),
        pltpu.VMEM((_CHUNK, _CG), jnp.float32),
        pltpu.VMEM((_ZCH, _CG), jnp.float32),
        pltpu.SemaphoreType.DMA,
        pltpu.SemaphoreType.DMA,
        pltpu.SemaphoreType.DMA,
        pltpu.SemaphoreType.DMA,
        pltpu.SemaphoreType.DMA,
        pltpu.SemaphoreType.DMA,
        pltpu.SemaphoreType.DMA,
        pltpu.SemaphoreType.DMA,
        pltpu.SemaphoreType.DMA,
    ],
    compiler_params=pltpu.CompilerParams(use_tc_tiling_on_sc=False),
)
def _scatter_phase(keys_hbm, feats_hbm, sums_hbm, counts_hbm,
                   table_sh, keys0, keys1, fbuf0, fbuf1, zbuf,
                   sk0, sk1, sf0, sf1, sw0, sw1, sz, ss0, ss1):
    c = lax.axis_index("c")
    s = lax.axis_index("s")
    keys_b, fbuf_b = [keys0, keys1], [fbuf0, fbuf1]
    sk, sf, sw, ss = [sk0, sk1], [sf0, sf1], [sw0, sw1], [ss0, ss1]

    def _fill(ref, n, val):
        def body(i, _):
            ref[i, :] = jnp.full((_CG,), val, jnp.float32)
            return 0
        lax.fori_loop(0, n, body, 0)

    _fill(zbuf, _ZCH, 0.0)

    zds = [
        pltpu.async_copy(zbuf, table_sh.at[pl.ds(s * _STRIPE + t * _ZCH, _ZCH)], sz)
        for t in range(_STRIPE // _ZCH)
    ]
    for zd in zds:
        zd.wait()

    _DCH = 1024

    def dump_and_rezero(dump_fn, rezero):
        wd = [None, None]
        zds = []
        for t in range(_STRIPE // _DCH):
            b = t % 2
            r0 = s * _STRIPE + t * _DCH
            if wd[b] is not None:
                wd[b].wait()
            pltpu.sync_copy(table_sh.at[pl.ds(r0, _DCH)], fbuf_b[b])
            wd[b] = dump_fn(r0, fbuf_b[b], sw[b])
            if rezero:
                for h in range(_DCH // _ZCH):
                    zds.append(pltpu.async_copy(
                        zbuf, table_sh.at[pl.ds(r0 + h * _ZCH, _ZCH)], sz))
        for d in wd:
            d.wait()
        for zd in zds:
            zd.wait()

    def start_loads(j, col):
        b = j % 2
        base = s * _RPT + j * _CHUNK
        kd = pltpu.async_copy(keys_hbm.at[pl.ds(base, _CHUNK)], keys_b[b], sk[b])
        fd = pltpu.async_copy(
            feats_hbm.at[pl.ds(base, _CHUNK), pl.ds(col, _CG)], fbuf_b[b], sf[b])
        return kd, fd

    for p in range(_GPC):
        col = (c * _GPC + p) * _CG
        plsc.subcore_barrier()
        pend = [None, None]
        sd = [None, None]
        pend[0] = start_loads(0, col)
        for j in range(_NCHUNK):
            b = j % 2
            pb = (j + 1) % 2
            kd, fd = pend[b]
            kd.wait()
            fd.wait()
            sd[b] = pltpu.async_copy(
                fbuf_b[b], table_sh.at[keys_b[b]], ss[b], add=True)
            if sd[pb] is not None:
                sd[pb].wait()
            if j + 1 < _NCHUNK:
                pend[pb] = start_loads(j + 1, col)
        sd[(_NCHUNK - 1) % 2].wait()
        plsc.subcore_barrier()

        def dump(r0, buf, sem, col=col):
            return pltpu.async_copy(
                buf, sums_hbm.at[pl.ds(r0, _DCH), pl.ds(col, _CG)], sem)

        dump_and_rezero(dump, rezero=True)

    _fill(fbuf0, _CHUNK, 1.0)
    plsc.subcore_barrier()
    kd = [None, None]
    sd = [None, None]
    kd[0] = pltpu.async_copy(
        keys_hbm.at[pl.ds(s * _RPT, _CHUNK)], keys_b[0], sk[0])
    for j in range(_NCHUNK):
        b = j % 2
        pb = (j + 1) % 2
        kd[b].wait()
        sd[b] = pltpu.async_copy(fbuf0, table_sh.at[keys_b[b]], ss[b], add=True)
        if sd[pb] is not None:
            sd[pb].wait()
        if j + 1 < _NCHUNK:
            base = s * _RPT + (j + 1) * _CHUNK
            kd[pb] = pltpu.async_copy(
                keys_hbm.at[pl.ds(base, _CHUNK)], keys_b[pb], sk[pb])
    sd[(_NCHUNK - 1) % 2].wait()
    plsc.subcore_barrier()

    @pl.when(c == 0)
    def _dump_counts():
        def dump(r0, buf, sem):
            return pltpu.async_copy(buf, counts_hbm.at[pl.ds(r0, _DCH)], sem)

        dump_and_rezero(dump, rezero=False)


_TBLK = 2048


def _div_body(sums_ref, counts_ref, out_ref):
    cnt = counts_ref[:, 0:1]
    out_ref[...] = sums_ref[...] / jnp.maximum(cnt, 1.0)


def _mean_table(sums, counts16):
    return pl.pallas_call(
        _div_body,
        grid=(_M // _TBLK,),
        in_specs=[
            pl.BlockSpec((_TBLK, _D), lambda i: (i, 0)),
            pl.BlockSpec((_TBLK, _CG), lambda i: (i, 0)),
        ],
        out_specs=pl.BlockSpec((_TBLK, _D), lambda i: (i, 0)),
        out_shape=jax.ShapeDtypeStruct((_M, _D), jnp.float32),
    )(sums, counts16)


_GB = _B // (_NC * _NS)
_GCH = 256


@functools.partial(
    pl.kernel,
    out_type=jax.ShapeDtypeStruct((_B, _D), jnp.float32),
    mesh=_mesh,
    scratch_types=[
        pltpu.VMEM((_GCH,), jnp.int32),
        pltpu.VMEM((_GCH,), jnp.int32),
        pltpu.VMEM((_GCH, _D), jnp.float32),
        pltpu.VMEM((_GCH, _D), jnp.float32),
        pltpu.SemaphoreType.DMA,
        pltpu.SemaphoreType.DMA,
        pltpu.SemaphoreType.DMA,
        pltpu.SemaphoreType.DMA,
        pltpu.SemaphoreType.DMA,
        pltpu.SemaphoreType.DMA,
    ],
    compiler_params=pltpu.CompilerParams(use_tc_tiling_on_sc=False),
)
def _gather_phase(keys_hbm, mean_hbm, out_hbm, idx0, idx1, rows0, rows1,
                  si0, si1, sg0, sg1, sw0, sw1):
    wid = lax.axis_index("s") * _NC + lax.axis_index("c")
    idx_b, rows_b = [idx0, idx1], [rows0, rows1]
    si, sg, sw = [si0, si1], [sg0, sg1], [sw0, sw1]
    nj = _GB // _GCH

    def chunk_base(j):
        return wid * _GB + j * _GCH

    kd = [None, None]
    gd = [None, None]
    wd = [None, None]
    kd[0] = pltpu.async_copy(keys_hbm.at[pl.ds(chunk_base(0), _GCH)], idx0, si0)
    for j in range(nj):
        b = j % 2
        pb = (j + 1) % 2
        if wd[b] is not None:
            wd[b].wait()
        kd[b].wait()
        gd[b] = pltpu.async_copy(mean_hbm.at[idx_b[b]], rows_b[b], sg[b])
        if j > 0:
            gd[pb].wait()
            wd[pb] = pltpu.async_copy(
                rows_b[pb], out_hbm.at[pl.ds(chunk_base(j - 1), _GCH)], sw[pb])
        if j + 1 < nj:
            kd[pb] = pltpu.async_copy(
                keys_hbm.at[pl.ds(chunk_base(j + 1), _GCH)], idx_b[pb], si[pb])
    last = (nj - 1) % 2
    gd[last].wait()
    wd[last] = pltpu.async_copy(
        rows_b[last], out_hbm.at[pl.ds(chunk_base(nj - 1), _GCH)], sw[last])
    wd[0].wait()
    wd[1].wait()


def kernel(style_ids, comp_ids, sc_feats):
    keys = style_ids.astype(jnp.int32) * _NCMP + comp_ids.astype(jnp.int32)
    sums, counts16 = _scatter_phase(keys, sc_feats)
    mean = _mean_table(sums, counts16)
    return _gather_phase(keys, mean)


# speedup vs baseline: 1.2376x; 1.2376x over previous
import functools

import jax
import jax.numpy as jnp
from jax import lax
from jax.experimental import pallas as pl
from jax.experimental.pallas import tpu as pltpu
from jax.experimental.pallas import tpu_sc as plsc

_NSTY = 256
_NCMP = 256
_B = 262144
_D = 128
_M = _NSTY * _NCMP

_NC, _NS = 2, 16
_CG = 16
_NG = _D // _CG
_GPC = _NG // _NC
_RPT = _B // _NS
_CHUNK = 1024
_NCHUNK = _RPT // _CHUNK
_STRIPE = _M // _NS
_ZCH = 512

_mesh = plsc.VectorSubcoreMesh(core_axis_name="c", subcore_axis_name="s")


@functools.partial(
    pl.kernel,
    out_type=[
        jax.ShapeDtypeStruct((_M, _D), jnp.float32),
        jax.ShapeDtypeStruct((_M,), jnp.float32),
    ],
    mesh=_mesh,
    scratch_types=[
        pltpu.VMEM_SHARED((_M, _CG), jnp.float32),
        pltpu.VMEM_SHARED((_M,), jnp.float32),
        pltpu.VMEM((_CHUNK,), jnp.int32),
        pltpu.VMEM((_CHUNK,), jnp.int32),
        pltpu.VMEM((_CHUNK, _CG), jnp.float32),
        pltpu.VMEM((_CHUNK, _CG), jnp.float32),
        pltpu.VMEM((_ZCH, _CG), jnp.float32),
        pltpu.VMEM((_CHUNK,), jnp.float32),
        pltpu.SemaphoreType.DMA,
        pltpu.SemaphoreType.DMA,
        pltpu.SemaphoreType.DMA,
        pltpu.SemaphoreType.DMA,
        pltpu.SemaphoreType.DMA,
        pltpu.SemaphoreType.DMA,
        pltpu.SemaphoreType.DMA,
    ],
    compiler_params=pltpu.CompilerParams(use_tc_tiling_on_sc=False),
)
def _scatter_phase(keys_hbm, feats_hbm, sums_hbm, counts_hbm,
                   table_sh, ctable_sh, keys0, keys1, fbuf0, fbuf1, zbuf,
                   ones_v, sk0, sk1, sf0, sf1, sw0, sw1, sz):
    c = lax.axis_index("c")
    s = lax.axis_index("s")
    keys_b, fbuf_b = [keys0, keys1], [fbuf0, fbuf1]
    sk, sf, sw = [sk0, sk1], [sf0, sf1], [sw0, sw1]

    def _fill(ref, n, val):
        def body(i, _):
            ref[i, :] = jnp.full((_CG,), val, jnp.float32)
            return 0
        lax.fori_loop(0, n, body, 0)

    def _fill1d(ref, n, val):
        def body(i, _):
            ref[pl.ds(i * _CG, _CG)] = jnp.full((_CG,), val, jnp.float32)
            return 0
        lax.fori_loop(0, n // _CG, body, 0)

    _fill(zbuf, _ZCH, 0.0)
    _fill1d(ones_v, _CHUNK, 0.0)

    zds = [
        pltpu.async_copy(zbuf, table_sh.at[pl.ds(s * _STRIPE + t * _ZCH, _ZCH)], sz)
        for t in range(_STRIPE // _ZCH)
    ]
    zds += [
        pltpu.async_copy(ones_v, ctable_sh.at[pl.ds(s * _STRIPE + t * _CHUNK, _CHUNK)], sz)
        for t in range(_STRIPE // _CHUNK)
    ]
    for zd in zds:
        zd.wait()
    _fill1d(ones_v, _CHUNK, 1.0)

    _DCH = 1024

    def dump_and_rezero(dump_fn, rezero):
        wd = [None, None]
        zds = []
        for t in range(_STRIPE // _DCH):
            b = t % 2
            r0 = s * _STRIPE + t * _DCH
            if wd[b] is not None:
                wd[b].wait()
            pltpu.sync_copy(table_sh.at[pl.ds(r0, _DCH)], fbuf_b[b])
            wd[b] = dump_fn(r0, fbuf_b[b], sw[b])
            if rezero:
                for h in range(_DCH // _ZCH):
                    zds.append(pltpu.async_copy(
                        zbuf, table_sh.at[pl.ds(r0 + h * _ZCH, _ZCH)], sz))
        for d in wd:
            d.wait()
        for zd in zds:
            zd.wait()

    def start_loads(j, col):
        b = j % 2
        base = s * _RPT + j * _CHUNK
        kd = pltpu.async_copy(keys_hbm.at[pl.ds(base, _CHUNK)], keys_b[b], sk[b])
        fd = pltpu.async_copy(
            feats_hbm.at[pl.ds(base, _CHUNK), pl.ds(col, _CG)], fbuf_b[b], sf[b])
        return kd, fd

    for p in range(_GPC):
        col = (c * _GPC + p) * _CG
        plsc.subcore_barrier()
        pend = [None, None]
        pend[0] = start_loads(0, col)
        for j in range(_NCHUNK):
            b = j % 2
            if j + 1 < _NCHUNK:
                pend[(j + 1) % 2] = start_loads(j + 1, col)
            kd, fd = pend[b]
            kd.wait()
            fd.wait()
            pltpu.sync_copy(fbuf_b[b], table_sh.at[keys_b[b]], add=True)
            if p == 0:
                @pl.when(c == 0)
                def _scat_ones(b=b):
                    pltpu.sync_copy(ones_v, ctable_sh.at[keys_b[b]], add=True)
        plsc.subcore_barrier()

        if p == 0:
            @pl.when(c == 0)
            def _dump_counts():
                pltpu.async_copy(
                    ctable_sh.at[pl.ds(s * _STRIPE, _STRIPE)],
                    counts_hbm.at[pl.ds(s * _STRIPE, _STRIPE)], sz).wait()

        def dump(r0, buf, sem, col=col):
            return pltpu.async_copy(
                buf, sums_hbm.at[pl.ds(r0, _DCH), pl.ds(col, _CG)], sem)

        dump_and_rezero(dump, rezero=(p + 1 < _GPC))


_TBLK = 2048


def _div_body(sums_ref, counts_ref, out_ref):
    cnt = counts_ref[...][:, None]
    out_ref[...] = sums_ref[...] / jnp.maximum(cnt, 1.0)


def _mean_table(sums, counts):
    return pl.pallas_call(
        _div_body,
        grid=(_M // _TBLK,),
        in_specs=[
            pl.BlockSpec((_TBLK, _D), lambda i: (i, 0)),
            pl.BlockSpec((_TBLK,), lambda i: (i,)),
        ],
        out_specs=pl.BlockSpec((_TBLK, _D), lambda i: (i, 0)),
        out_shape=jax.ShapeDtypeStruct((_M, _D), jnp.float32),
    )(sums, counts)


_GB = _B // (_NC * _NS)
_GCH = 256


@functools.partial(
    pl.kernel,
    out_type=jax.ShapeDtypeStruct((_B, _D), jnp.float32),
    mesh=_mesh,
    scratch_types=[
        pltpu.VMEM((_GCH,), jnp.int32),
        pltpu.VMEM((_GCH,), jnp.int32),
        pltpu.VMEM((_GCH, _D), jnp.float32),
        pltpu.VMEM((_GCH, _D), jnp.float32),
        pltpu.SemaphoreType.DMA,
        pltpu.SemaphoreType.DMA,
        pltpu.SemaphoreType.DMA,
        pltpu.SemaphoreType.DMA,
        pltpu.SemaphoreType.DMA,
        pltpu.SemaphoreType.DMA,
    ],
    compiler_params=pltpu.CompilerParams(use_tc_tiling_on_sc=False),
)
def _gather_phase(keys_hbm, mean_hbm, out_hbm, idx0, idx1, rows0, rows1,
                  si0, si1, sg0, sg1, sw0, sw1):
    wid = lax.axis_index("s") * _NC + lax.axis_index("c")
    idx_b, rows_b = [idx0, idx1], [rows0, rows1]
    si, sg, sw = [si0, si1], [sg0, sg1], [sw0, sw1]
    nj = _GB // _GCH

    def chunk_base(j):
        return wid * _GB + j * _GCH

    kd = [None, None]
    gd = [None, None]
    wd = [None, None]
    kd[0] = pltpu.async_copy(keys_hbm.at[pl.ds(chunk_base(0), _GCH)], idx0, si0)
    for j in range(nj):
        b = j % 2
        pb = (j + 1) % 2
        if wd[b] is not None:
            wd[b].wait()
        kd[b].wait()
        gd[b] = pltpu.async_copy(mean_hbm.at[idx_b[b]], rows_b[b], sg[b])
        if j > 0:
            gd[pb].wait()
            wd[pb] = pltpu.async_copy(
                rows_b[pb], out_hbm.at[pl.ds(chunk_base(j - 1), _GCH)], sw[pb])
        if j + 1 < nj:
            kd[pb] = pltpu.async_copy(
                keys_hbm.at[pl.ds(chunk_base(j + 1), _GCH)], idx_b[pb], si[pb])
    last = (nj - 1) % 2
    gd[last].wait()
    wd[last] = pltpu.async_copy(
        rows_b[last], out_hbm.at[pl.ds(chunk_base(nj - 1), _GCH)], sw[last])
    wd[0].wait()
    wd[1].wait()


def kernel(style_ids, comp_ids, sc_feats):
    keys = style_ids.astype(jnp.int32) * _NCMP + comp_ids.astype(jnp.int32)
    sums, counts = _scatter_phase(keys, sc_feats)
    mean = _mean_table(sums, counts)
    return _gather_phase(keys, mean)


# speedup vs baseline: 1.2421x; 1.0036x over previous
import functools

import jax
import jax.numpy as jnp
from jax import lax
from jax.experimental import pallas as pl
from jax.experimental.pallas import tpu as pltpu
from jax.experimental.pallas import tpu_sc as plsc

_NSTY = 256
_NCMP = 256
_B = 262144
_D = 128
_M = _NSTY * _NCMP

_NC, _NS = 2, 16
_CG = 16
_NG = _D // _CG
_GPC = _NG // _NC
_RPT = _B // _NS
_CHUNK = 1024
_NCHUNK = _RPT // _CHUNK
_STRIPE = _M // _NS
_ZCH = 512

_mesh = plsc.VectorSubcoreMesh(core_axis_name="c", subcore_axis_name="s")


@functools.partial(
    pl.kernel,
    out_type=[
        jax.ShapeDtypeStruct((_M, _D), jnp.float32),
        jax.ShapeDtypeStruct((_M,), jnp.float32),
    ],
    mesh=_mesh,
    scratch_types=[
        pltpu.VMEM_SHARED((_M, _CG), jnp.float32),
        pltpu.VMEM_SHARED((_M,), jnp.float32),
        pltpu.VMEM((_CHUNK,), jnp.int32),
        pltpu.VMEM((_CHUNK,), jnp.int32),
        pltpu.VMEM((_CHUNK, _CG), jnp.float32),
        pltpu.VMEM((_CHUNK, _CG), jnp.float32),
        pltpu.VMEM((_ZCH, _CG), jnp.float32),
        pltpu.VMEM((_CHUNK,), jnp.float32),
        pltpu.SemaphoreType.DMA,
        pltpu.SemaphoreType.DMA,
        pltpu.SemaphoreType.DMA,
        pltpu.SemaphoreType.DMA,
        pltpu.SemaphoreType.DMA,
        pltpu.SemaphoreType.DMA,
        pltpu.SemaphoreType.DMA,
    ],
    compiler_params=pltpu.CompilerParams(use_tc_tiling_on_sc=False),
)
def _scatter_phase(keys_hbm, feats_hbm, sums_hbm, counts_hbm,
                   table_sh, ctable_sh, keys0, keys1, fbuf0, fbuf1, zbuf,
                   ones_v, sk0, sk1, sf0, sf1, sw0, sw1, sz):
    c = lax.axis_index("c")
    s = lax.axis_index("s")
    keys_b, fbuf_b = [keys0, keys1], [fbuf0, fbuf1]
    sk, sf, sw = [sk0, sk1], [sf0, sf1], [sw0, sw1]

    def _fill(ref, n, val):
        def body(i, _):
            ref[i, :] = jnp.full((_CG,), val, jnp.float32)
            return 0
        lax.fori_loop(0, n, body, 0)

    def _fill1d(ref, n, val):
        def body(i, _):
            ref[pl.ds(i * _CG, _CG)] = jnp.full((_CG,), val, jnp.float32)
            return 0
        lax.fori_loop(0, n // _CG, body, 0)

    _fill(zbuf, _ZCH, 0.0)
    _fill1d(ones_v, _CHUNK, 0.0)

    zds = [
        pltpu.async_copy(zbuf, table_sh.at[pl.ds(s * _STRIPE + t * _ZCH, _ZCH)], sz)
        for t in range(_STRIPE // _ZCH)
    ]
    zds += [
        pltpu.async_copy(ones_v, ctable_sh.at[pl.ds(s * _STRIPE + t * _CHUNK, _CHUNK)], sz)
        for t in range(_STRIPE // _CHUNK)
    ]
    for zd in zds:
        zd.wait()
    _fill1d(ones_v, _CHUNK, 1.0)

    _DCH = 1024

    def dump_and_rezero(dump_fn, rezero):
        wd = [None, None]
        zds = []
        for t in range(_STRIPE // _DCH):
            b = t % 2
            r0 = s * _STRIPE + t * _DCH
            if wd[b] is not None:
                wd[b].wait()
            pltpu.sync_copy(table_sh.at[pl.ds(r0, _DCH)], fbuf_b[b])
            wd[b] = dump_fn(r0, fbuf_b[b], sw[b])
            if rezero:
                for h in range(_DCH // _ZCH):
                    zds.append(pltpu.async_copy(
                        zbuf, table_sh.at[pl.ds(r0 + h * _ZCH, _ZCH)], sz))
        for d in wd:
            d.wait()
        for zd in zds:
            zd.wait()

    def start_loads(j, col):
        b = j % 2
        base = s * _RPT + j * _CHUNK
        kd = pltpu.async_copy(keys_hbm.at[pl.ds(base, _CHUNK)], keys_b[b], sk[b])
        fd = pltpu.async_copy(
            feats_hbm.at[pl.ds(base, _CHUNK), pl.ds(col, _CG)], fbuf_b[b], sf[b])
        return kd, fd

    for p in range(_GPC):
        col = (c * _GPC + p) * _CG
        plsc.subcore_barrier()
        pend = [None, None]
        pend[0] = start_loads(0, col)
        for j in range(_NCHUNK):
            b = j % 2
            if j + 1 < _NCHUNK:
                pend[(j + 1) % 2] = start_loads(j + 1, col)
            kd, fd = pend[b]
            kd.wait()
            fd.wait()
            pltpu.sync_copy(fbuf_b[b], table_sh.at[keys_b[b]], add=True)
            if p == 0:
                @pl.when(c == 0)
                def _scat_ones(b=b):
                    pltpu.sync_copy(ones_v, ctable_sh.at[keys_b[b]], add=True)
        plsc.subcore_barrier()

        if p == 0:
            @pl.when(c == 0)
            def _dump_counts():
                pltpu.async_copy(
                    ctable_sh.at[pl.ds(s * _STRIPE, _STRIPE)],
                    counts_hbm.at[pl.ds(s * _STRIPE, _STRIPE)], sz).wait()

        def dump(r0, buf, sem, col=col):
            return pltpu.async_copy(
                buf, sums_hbm.at[pl.ds(r0, _DCH), pl.ds(col, _CG)], sem)

        dump_and_rezero(dump, rezero=(p + 1 < _GPC))


_TBLK = 2048


def _div_body(sums_ref, counts_ref, out_ref):
    cnt = counts_ref[...][:, None]
    out_ref[...] = sums_ref[...] / jnp.maximum(cnt, 1.0)


def _mean_table(sums, counts):
    return pl.pallas_call(
        _div_body,
        grid=(_M // _TBLK,),
        in_specs=[
            pl.BlockSpec((_TBLK, _D), lambda i: (i, 0)),
            pl.BlockSpec((_TBLK,), lambda i: (i,)),
        ],
        out_specs=pl.BlockSpec((_TBLK, _D), lambda i: (i, 0)),
        out_shape=jax.ShapeDtypeStruct((_M, _D), jnp.float32),
    )(sums, counts)


_GB = _B // (_NC * _NS)
_GCH = 256


@functools.partial(
    pl.kernel,
    out_type=jax.ShapeDtypeStruct((_B, _D), jnp.float32),
    mesh=_mesh,
    scratch_types=[
        pltpu.VMEM((_GB,), jnp.int32),
        pltpu.VMEM((_GCH, _D), jnp.float32),
        pltpu.VMEM((_GCH, _D), jnp.float32),
        pltpu.SemaphoreType.DMA,
        pltpu.SemaphoreType.DMA,
        pltpu.SemaphoreType.DMA,
        pltpu.SemaphoreType.DMA,
        pltpu.SemaphoreType.DMA,
    ],
    compiler_params=pltpu.CompilerParams(use_tc_tiling_on_sc=False),
)
def _gather_phase(keys_hbm, mean_hbm, out_hbm, idx_all, rows0, rows1,
                  si, sg0, sg1, sw0, sw1):
    wid = lax.axis_index("s") * _NC + lax.axis_index("c")
    rows_b = [rows0, rows1]
    sg, sw = [sg0, sg1], [sw0, sw1]
    nj = _GB // _GCH

    def chunk_base(j):
        return wid * _GB + j * _GCH

    pltpu.sync_copy(keys_hbm.at[pl.ds(chunk_base(0), _GB)], idx_all)
    gd = [None, None]
    wd = [None, None]
    for j in range(nj):
        b = j % 2
        pb = (j + 1) % 2
        if wd[b] is not None:
            wd[b].wait()
        gd[b] = pltpu.async_copy(
            mean_hbm.at[idx_all.at[pl.ds(j * _GCH, _GCH)]], rows_b[b], sg[b])
        if j > 0:
            gd[pb].wait()
            wd[pb] = pltpu.async_copy(
                rows_b[pb], out_hbm.at[pl.ds(chunk_base(j - 1), _GCH)], sw[pb])
    last = (nj - 1) % 2
    gd[last].wait()
    wd[last] = pltpu.async_copy(
        rows_b[last], out_hbm.at[pl.ds(chunk_base(nj - 1), _GCH)], sw[last])
    wd[0].wait()
    wd[1].wait()


def kernel(style_ids, comp_ids, sc_feats):
    keys = style_ids.astype(jnp.int32) * _NCMP + comp_ids.astype(jnp.int32)
    sums, counts = _scatter_phase(keys, sc_feats)
    mean = _mean_table(sums, counts)
    return _gather_phase(keys, mean)
